# Initial kernel scaffold; baseline (speedup 1.0000x reference)
import jax
import jax.numpy as jnp
from jax.experimental import pallas as pl


def kernel(x, edge_index, batch, conv_W0, conv_b0, conv_W1, conv_b1, conv_W2, conv_b2, mlp_W0, mlp_b0, mlp_W1, mlp_b1, mlp_W2, mlp_b2, mlp_W3, mlp_b3, bn_g0, bn_b0, bn_g1, bn_b1, bn_g2, bn_b2):
    raise NotImplementedError("write your pallas kernel here")


# speedup vs baseline: 6.5463x; 6.5463x over previous
import jax
import jax.numpy as jnp
from jax import lax
from jax.experimental import pallas as pl
from jax.experimental.pallas import tpu as pltpu
from jax.experimental.pallas import tpu_sc as plsc

N = 10000
E = 320000
DX = 128
HC = 64
G = 16

SC_CORES = 2
SC_TILES = 16
NW = SC_CORES * SC_TILES
CH = 128
EPAD = ((E + NW * CH - 1) // (NW * CH)) * (NW * CH)
EPW = EPAD // NW
NCHUNK = EPW // CH
R = 10240
RPT = R // SC_TILES

_HI = lax.Precision.HIGHEST


def _mesh():
    return plsc.VectorSubcoreMesh(
        core_axis_name="c", subcore_axis_name="s",
        num_cores=SC_CORES, num_subcores=SC_TILES)


def _msg_body(hs_hbm, src_hbm, dst_hbm, zeros128_hbm, out_hbm,
              sidx_v, didx_v, rows_v, acc_sh, sem):
    c = lax.axis_index("c")
    s = lax.axis_index("s")
    wid = c * SC_TILES + s
    pltpu.sync_copy(zeros128_hbm, rows_v)
    for k in range(RPT // CH):
        pltpu.sync_copy(rows_v, acc_sh.at[pl.ds(s * RPT + k * CH, CH)])
    plsc.subcore_barrier()

    def chunk(k, carry):
        base = wid * EPW + k * CH
        pltpu.sync_copy(src_hbm.at[pl.ds(base, CH)], sidx_v)
        pltpu.async_copy(hs_hbm.at[sidx_v], rows_v, sem).wait()
        pltpu.sync_copy(dst_hbm.at[pl.ds(base, CH)], didx_v)
        pltpu.sync_copy(rows_v, acc_sh.at[didx_v], add=True)
        return carry

    lax.fori_loop(0, NCHUNK, chunk, 0)
    plsc.subcore_barrier()
    for k in range(RPT // CH):
        r0 = s * RPT + k * CH
        pltpu.sync_copy(acc_sh.at[pl.ds(r0, CH)], rows_v)
        pltpu.sync_copy(rows_v, out_hbm.at[c, pl.ds(r0, CH)])


def _sc_msg(hs, srcp, dstp, zeros128):
    return pl.kernel(
        _msg_body,
        out_type=jax.ShapeDtypeStruct((SC_CORES, R, DX), jnp.float32),
        mesh=_mesh(),
        scratch_types=[
            pltpu.VMEM((CH,), jnp.int32),
            pltpu.VMEM((CH,), jnp.int32),
            pltpu.VMEM((CH, DX), jnp.float32),
            pltpu.VMEM_SHARED((R, DX), jnp.float32),
            pltpu.SemaphoreType.DMA,
        ],
    )(hs, srcp, dstp, zeros128)


def _onehot_and_cnt(batch2d):
    onehot = (batch2d[...] == lax.broadcasted_iota(jnp.int32, (G, N), 0)
              ).astype(jnp.float32)
    cnt = jnp.sum(onehot, axis=1, keepdims=True)
    return onehot, jnp.maximum(cnt, 1.0)


def _tc0_body(x, batch2d, deg0, deg1, W0, dinv_o, hs0_o, xinit_o):
    dinv = lax.rsqrt(deg0[...] + deg1[...] + 1.0)
    dinv_o[...] = dinv
    hs0 = jnp.dot(x[...], W0[...], preferred_element_type=jnp.float32,
                  precision=_HI) * dinv
    hs0_o[...] = jnp.pad(hs0, ((0, R - N), (0, DX - HC)))
    onehot, cnt = _onehot_and_cnt(batch2d)
    xinit_o[...] = jnp.dot(onehot, x[...], preferred_element_type=jnp.float32,
                           precision=_HI) / cnt


def _tc0(x, batch2d, deg0, deg1, W0):
    return pl.pallas_call(
        _tc0_body,
        out_shape=[
            jax.ShapeDtypeStruct((N, 1), jnp.float32),
            jax.ShapeDtypeStruct((R, DX), jnp.float32),
            jax.ShapeDtypeStruct((G, DX), jnp.float32),
        ],
    )(x, batch2d, deg0, deg1, W0)


def _tc_layer_body(p0, p1, hs, dinv, b, Wn, batch2d, hsn_o, pooled_o):
    xn = jnp.tanh(dinv[...] * (p0[...] + p1[...] + hs[...]) + b[...])
    onehot, cnt = _onehot_and_cnt(batch2d)
    pooled_o[...] = jnp.dot(onehot, xn, preferred_element_type=jnp.float32,
                            precision=_HI) / cnt
    hsn_o[...] = jnp.pad(jnp.dot(xn, Wn[...], preferred_element_type=jnp.float32,
                                 precision=_HI) * dinv[...], ((0, R - N), (0, DX - HC)))


def _tc_layer(p0, p1, hs, dinv, b, Wn, batch2d):
    return pl.pallas_call(
        _tc_layer_body,
        out_shape=[
            jax.ShapeDtypeStruct((R, DX), jnp.float32),
            jax.ShapeDtypeStruct((G, HC), jnp.float32),
        ],
    )(p0, p1, hs, dinv, b, Wn, batch2d)


def _bn_relu(h, g, b):
    mu = jnp.mean(h, axis=0, keepdims=True)
    var = jnp.mean((h - mu) ** 2, axis=0, keepdims=True)
    return jnp.maximum((h - mu) * lax.rsqrt(var + 1e-5) * g + b, 0.0)


def _tc_final_body(p0, p1, hs, dinv, b2, batch2d, x_init, pooled1, pooled2,
                   mW0, mb0, mW1, mb1, mW2, mb2, mW3, mb3,
                   g0, bb0, g1, bb1, g2, bb2, out_o):
    xn = jnp.tanh(dinv[...] * (p0[...] + p1[...] + hs[...]) + b2[...])
    onehot, cnt = _onehot_and_cnt(batch2d)
    pooled3 = jnp.dot(onehot, xn, preferred_element_type=jnp.float32,
                      precision=_HI) / cnt
    h = jnp.concatenate([x_init[...], pooled1[...], pooled2[...], pooled3],
                        axis=1)
    h = _bn_relu(jnp.dot(h, mW0[...], preferred_element_type=jnp.float32,
                         precision=_HI) + mb0[...], g0[...], bb0[...])
    h = _bn_relu(jnp.dot(h, mW1[...], preferred_element_type=jnp.float32,
                         precision=_HI) + mb1[...], g1[...], bb1[...])
    h = _bn_relu(jnp.dot(h, mW2[...], preferred_element_type=jnp.float32,
                         precision=_HI) + mb2[...], g2[...], bb2[...])
    o = jnp.dot(h, mW3[...], preferred_element_type=jnp.float32,
                precision=_HI) + mb3[...]
    m = jnp.max(o, axis=1, keepdims=True)
    lse = m + jnp.log(jnp.sum(jnp.exp(o - m), axis=1, keepdims=True))
    out_o[...] = o - lse


def _tc_final(*args):
    return pl.pallas_call(
        _tc_final_body,
        out_shape=jax.ShapeDtypeStruct((G, 2), jnp.float32),
    )(*args)


def kernel(x, edge_index, batch, conv_W0, conv_b0, conv_W1, conv_b1,
           conv_W2, conv_b2, mlp_W0, mlp_b0, mlp_W1, mlp_b1, mlp_W2, mlp_b2,
           mlp_W3, mlp_b3, bn_g0, bn_b0, bn_g1, bn_b1, bn_g2, bn_b2):
    src = edge_index[0]
    dst = edge_index[1]
    pad = EPAD - E
    srcp = jnp.concatenate([src, jnp.zeros((pad,), src.dtype)])
    dstp = jnp.concatenate([dst, jnp.full((pad,), N, dst.dtype)])
    zeros128 = jnp.zeros((CH, DX), jnp.float32)
    ones_r = jnp.ones((R, DX), jnp.float32)
    batch2d = batch.reshape(1, N)

    degparts = _sc_msg(ones_r, srcp, dstp, zeros128)
    deg0 = degparts[0, :N, 0:1]
    deg1 = degparts[1, :N, 0:1]

    dinv, hs0, x_init = _tc0(x, batch2d, deg0, deg1, conv_W0)

    parts = _sc_msg(hs0, srcp, dstp, zeros128)
    hs1, pooled1 = _tc_layer(parts[0, :N, :HC], parts[1, :N, :HC], hs0[:N, :HC], dinv,
                             conv_b0.reshape(1, HC), conv_W1, batch2d)
    parts = _sc_msg(hs1, srcp, dstp, zeros128)
    hs2, pooled2 = _tc_layer(parts[0, :N, :HC], parts[1, :N, :HC], hs1[:N, :HC], dinv,
                             conv_b1.reshape(1, HC), conv_W2, batch2d)
    parts = _sc_msg(hs2, srcp, dstp, zeros128)

    return _tc_final(parts[0, :N, :HC], parts[1, :N, :HC], hs2[:N, :HC], dinv,
                     conv_b2.reshape(1, HC), batch2d, x_init, pooled1, pooled2,
                     mlp_W0, mlp_b0.reshape(1, -1), mlp_W1, mlp_b1.reshape(1, -1),
                     mlp_W2, mlp_b2.reshape(1, -1), mlp_W3, mlp_b3.reshape(1, -1),
                     bn_g0.reshape(1, -1), bn_b0.reshape(1, -1),
                     bn_g1.reshape(1, -1), bn_b1.reshape(1, -1),
                     bn_g2.reshape(1, -1), bn_b2.reshape(1, -1))
